# baseline (speedup 1.0000x reference)
import functools

import jax
import jax.numpy as jnp
from jax import lax
from jax.experimental import pallas as pl
from jax.experimental.pallas import tpu as pltpu
from jax.experimental.pallas import tpu_sc as plsc

N = 10000
E = 320000
N_ATOM = 8
N_NODE = 128
N_EDGE = 16
N_BASIS = 8
HID = 64
R_MAX = 5.0

NC = 2
NS = 16
NW = NC * NS
CHUNK = 80
SUPER = 5
PARTS = 5
PART = E // PARTS
PW = PART // NW
PITERS = PW // CHUNK

_f32 = jnp.float32


def _sc_mesh():
    return plsc.VectorSubcoreMesh(core_axis_name="c", subcore_axis_name="s")


def _dual_gather(table_a, table_b, idx_a, idx_b, name):
    D = table_a.shape[1]

    @functools.partial(
        pl.kernel,
        out_type=(
            jax.ShapeDtypeStruct((PART, D), _f32),
            jax.ShapeDtypeStruct((PART, D), _f32),
        ),
        mesh=_sc_mesh(),
        scratch_types=[
            [pltpu.VMEM((CHUNK,), jnp.int32)] * 2,
            [pltpu.VMEM((CHUNK, D), _f32)] * 2,
            pltpu.SemaphoreType.DMA,
            pltpu.SemaphoreType.DMA,
        ],
        name=name,
    )
    def k(ta, tb, ia, ib, oa, ob, idxv, rowv, sema, semb):
        wid = lax.axis_index("s") * NC + lax.axis_index("c")
        base = wid * PW
        iav, ibv = idxv
        rav, rbv = rowv

        def body(i, carry):
            b = base + i * CHUNK
            pltpu.sync_copy(ia.at[pl.ds(b, CHUNK)], iav)
            pltpu.sync_copy(ib.at[pl.ds(b, CHUNK)], ibv)
            cpa = pltpu.async_copy(ta.at[iav], rav, sema)
            cpb = pltpu.async_copy(tb.at[ibv], rbv, semb)
            cpa.wait()
            cpb.wait()
            pltpu.sync_copy(rav, oa.at[pl.ds(b, CHUNK)])
            pltpu.sync_copy(rbv, ob.at[pl.ds(b, CHUNK)])
            return carry

        lax.fori_loop(0, PITERS, body, 0)

    return k(table_a, table_b, idx_a, idx_b)


def _atom_gather(atoms, idx_a, idx_b, name):
    @functools.partial(
        pl.kernel,
        out_type=(
            jax.ShapeDtypeStruct((PART,), jnp.int32),
            jax.ShapeDtypeStruct((PART,), jnp.int32),
        ),
        mesh=_sc_mesh(),
        scratch_types=[
            [pltpu.VMEM((CHUNK,), jnp.int32)] * 4,
            pltpu.SemaphoreType.DMA,
            pltpu.SemaphoreType.DMA,
        ],
        name=name,
    )
    def k(tab, ia, ib, oa, ob, bufs, sema, semb):
        wid = lax.axis_index("s") * NC + lax.axis_index("c")
        base = wid * PW
        iav, ibv, rav, rbv = bufs

        def body(i, carry):
            b = base + i * CHUNK
            pltpu.sync_copy(ia.at[pl.ds(b, CHUNK)], iav)
            pltpu.sync_copy(ib.at[pl.ds(b, CHUNK)], ibv)
            cpa = pltpu.async_copy(tab.at[iav], rav, sema)
            cpb = pltpu.async_copy(tab.at[ibv], rbv, semb)
            cpa.wait()
            cpb.wait()
            pltpu.sync_copy(rav, oa.at[pl.ds(b, CHUNK)])
            pltpu.sync_copy(rbv, ob.at[pl.ds(b, CHUNK)])
            return carry

        lax.fori_loop(0, PITERS, body, 0)

    return k(atoms, idx_a, idx_b)


NPAD = 10240


def _scatter_add(msg_parts, idx_dst, zeros_nf):
    rows_pt = NPAD // NS

    @functools.partial(
        pl.kernel,
        out_type=jax.ShapeDtypeStruct((NC, NPAD, N_NODE), _f32),
        mesh=_sc_mesh(),
        scratch_types=[
            pltpu.VMEM((CHUNK,), jnp.int32),
            pltpu.VMEM((CHUNK, N_NODE), _f32),
            pltpu.VMEM_SHARED((NPAD, N_NODE), _f32),
        ],
        name="sc_scatter_add",
    )
    def k(m0, m1, m2, m3, m4, idx_h, zero_h, out_h, idxv, msgv, acc):
        cid = lax.axis_index("c")
        sid = lax.axis_index("s")
        wid = sid * NC + cid
        pltpu.sync_copy(zero_h.at[pl.ds(sid * rows_pt, rows_pt)],
                        acc.at[pl.ds(sid * rows_pt, rows_pt)])
        plsc.subcore_barrier()

        for p, msg_h in enumerate((m0, m1, m2, m3, m4)):
            def body(i, carry, msg_h=msg_h, p=p):
                bl = wid * PW + i * CHUNK
                pltpu.sync_copy(idx_h.at[pl.ds(p * PART + bl, CHUNK)], idxv)
                pltpu.sync_copy(msg_h.at[pl.ds(bl, CHUNK)], msgv)
                pltpu.sync_copy(msgv, acc.at[idxv], add=True)
                return carry

            lax.fori_loop(0, PITERS, body, 0)
        plsc.subcore_barrier()
        pltpu.sync_copy(acc.at[pl.ds(sid * rows_pt, rows_pt)],
                        out_h.at[cid, pl.ds(sid * rows_pt, rows_pt)])

    return k(*msg_parts, idx_dst, zeros_nf)


def _sigmoid(x):
    return jax.nn.sigmoid(x)


def _softplus(x):
    return jnp.maximum(x, 0.0) + jnp.log1p(jnp.exp(-jnp.abs(x)))


_RN = 12582912.0
_SIN_C = [6.283185302468587, -41.34170039298465, 81.6051308111703,
          -76.70300555852963, 42.02585149179802, -14.899578443268958,
          3.2377800937143286]


def _fast_recip(y):
    i = lax.bitcast_convert_type(y, jnp.int32)
    r = lax.bitcast_convert_type(0x7EF311C3 - i, _f32)
    r = r * (2.0 - y * r)
    r = r * (2.0 - y * r)
    r = r * (2.0 - y * r)
    return r


def _bessel(r, bw):
    u = bw * (r * float(1.0 / (2.0 * 3.141592653589793 * R_MAX)))
    n = (u + _RN) - _RN
    f = u - n
    f2 = f * f
    acc = f2 * _SIN_C[-1] + _SIN_C[-2]
    for v in _SIN_C[-3::-1]:
        acc = acc * f2 + v
    return (2.0 / R_MAX) ** 0.5 * (f * acc) * _fast_recip(r)


def _dot(a, b):
    return jax.lax.dot_general(a, b, (((1,), (0,)), ((), ())),
                               preferred_element_type=_f32)


def _node_precompute(atoms2d, W_node, b_node, W_e1, W_nf, W_ns):
    def body(at_ref, wn_ref, bn_ref, we1_ref, wnf_ref, wns_ref,
             nf_ref, u_ref, v_ref, ts8_ref, td8_ref):
        at = at_ref[...]
        oh = (at == lax.broadcasted_iota(jnp.int32, (N, N_ATOM), 1)).astype(_f32)
        nf = _dot(oh, wn_ref[...]) + bn_ref[...]
        nf_ref[...] = nf
        we1 = we1_ref[...]
        u_ref[...] = _dot(nf, we1[0:N_NODE, :])
        v_ref[...] = _dot(nf, we1[N_NODE:2 * N_NODE, :])
        nf8 = wn_ref[...] + bn_ref[...]
        wnf = wnf_ref[...]
        wns = wns_ref[...]
        ts8_ref[:, 0:N_NODE] = _dot(nf8, wnf[0:N_NODE, :])
        ts8_ref[:, N_NODE:2 * N_NODE] = _dot(nf8, wns[0:N_NODE, :])
        ts8_ref[:, 2 * N_NODE:] = _dot(nf8, we1[0:N_NODE, :])
        td8_ref[:, 0:N_NODE] = _dot(nf8, wnf[N_NODE:2 * N_NODE, :])
        td8_ref[:, N_NODE:2 * N_NODE] = _dot(nf8, wns[N_NODE:2 * N_NODE, :])
        td8_ref[:, 2 * N_NODE:] = _dot(nf8, we1[N_NODE:2 * N_NODE, :])

    return pl.pallas_call(
        body,
        out_shape=(
            jax.ShapeDtypeStruct((N, N_NODE), _f32),
            jax.ShapeDtypeStruct((N, HID), _f32),
            jax.ShapeDtypeStruct((N, HID), _f32),
            jax.ShapeDtypeStruct((N_ATOM, HID + 2 * N_NODE), _f32),
            jax.ShapeDtypeStruct((N_ATOM, HID + 2 * N_NODE), _f32),
        ),
        name="tc_node_precompute",
    )(atoms2d, W_node, b_node, W_e1, W_nf, W_ns)


BE = 4000
GRID_P = PART // BE


def _env_msg(at_s, at_d, env_len, ts8, td8, W_e1, b_e1, W_e2, b_e2,
             W_nf, b_nf, W_ns, b_ns, ln_g, ln_b, bw):
    def body(ats_ref, atd_ref, len_ref, ts8_ref, td8_ref, we1_ref, be1_ref,
             we2_ref, be2_ref, wnf_ref, bnf_ref, wns_ref, bns_ref,
             g_ref, bln_ref, bw_ref, msg_ref):
        ohs = (ats_ref[...] == lax.broadcasted_iota(
            jnp.int32, (BE, N_ATOM), 1)).astype(_f32)
        ohd = (atd_ref[...] == lax.broadcasted_iota(
            jnp.int32, (BE, N_ATOM), 1)).astype(_f32)
        fs = _dot(ohs, ts8_ref[...])
        fd = _dot(ohd, td8_ref[...])
        bes = _bessel(len_ref[...], bw_ref[...])
        we1 = we1_ref[...]
        pre = (fs[:, 2 * N_NODE:] + fd[:, 2 * N_NODE:]
               + _dot(bes, we1[2 * N_NODE:, :]) + be1_ref[...])
        h = pre * _sigmoid(pre)
        env_f = _dot(h, we2_ref[...]) + be2_ref[...]
        a = (fs[:, 0:N_NODE] + fd[:, 0:N_NODE]
             + _dot(env_f, wnf_ref[...][2 * N_NODE:, :]) + bnf_ref[...])
        c = (fs[:, N_NODE:2 * N_NODE] + fd[:, N_NODE:2 * N_NODE]
             + _dot(env_f, wns_ref[...][2 * N_NODE:, :]) + bns_ref[...])
        msg = _sigmoid(a) * _softplus(c)
        mu = jnp.mean(msg, axis=-1, keepdims=True)
        xc = msg - mu
        var = jnp.mean(xc * xc, axis=-1, keepdims=True)
        msg_ref[...] = xc * jax.lax.rsqrt(var + 1e-5) * g_ref[...] + bln_ref[...]

    wspec = lambda shape: pl.BlockSpec(shape, lambda i: (0, 0))
    return pl.pallas_call(
        body,
        grid=(GRID_P,),
        in_specs=[
            pl.BlockSpec((BE, 1), lambda i: (i, 0)),
            pl.BlockSpec((BE, 1), lambda i: (i, 0)),
            pl.BlockSpec((BE, 1), lambda i: (i, 0)),
            wspec(ts8.shape), wspec(td8.shape),
            wspec(W_e1.shape), wspec(b_e1.shape),
            wspec(W_e2.shape), wspec(b_e2.shape),
            wspec(W_nf.shape), wspec(b_nf.shape),
            wspec(W_ns.shape), wspec(b_ns.shape),
            wspec(ln_g.shape), wspec(ln_b.shape), wspec(bw.shape),
        ],
        out_specs=pl.BlockSpec((BE, N_NODE), lambda i: (i, 0)),
        out_shape=jax.ShapeDtypeStruct((PART, N_NODE), _f32),
        name="tc_env_msg",
    )(at_s, at_d, env_len, ts8, td8, W_e1, b_e1, W_e2, b_e2, W_nf, b_nf,
      W_ns, b_ns, ln_g, ln_b, bw)


def _node_update(node_f, acc, W_ef, W_es, U, V):
    def body(nf_ref, acc_ref, wef_ref, wes_ref, u_ref, v_ref,
             out_ref, ts_ref, td_ref):
        accv = acc_ref[...]
        nf = nf_ref[...] + accv[0, 0:N, :] + accv[1, 0:N, :]
        out_ref[...] = nf
        wef = wef_ref[...]
        wes = wes_ref[...]
        ts_ref[:, 0:HID] = u_ref[...]
        ts_ref[:, HID:HID + N_EDGE] = _dot(nf, wef[0:N_NODE, :])
        ts_ref[:, HID + N_EDGE:HID + 2 * N_EDGE] = _dot(nf, wes[0:N_NODE, :])
        ts_ref[:, HID + 2 * N_EDGE:] = jnp.zeros((N, 32), _f32)
        td_ref[:, 0:HID] = v_ref[...]
        td_ref[:, HID:HID + N_EDGE] = _dot(nf, wef[N_NODE + N_EDGE:, :])
        td_ref[:, HID + N_EDGE:HID + 2 * N_EDGE] = _dot(nf, wes[N_NODE + N_EDGE:, :])
        td_ref[:, HID + 2 * N_EDGE:] = jnp.zeros((N, 32), _f32)

    return pl.pallas_call(
        body,
        out_shape=(
            jax.ShapeDtypeStruct((N, N_NODE), _f32),
            jax.ShapeDtypeStruct((N, N_NODE), _f32),
            jax.ShapeDtypeStruct((N, N_NODE), _f32),
        ),
        name="tc_node_update",
    )(node_f, acc, W_ef, W_es, U, V)


def _edge_final(gEs, gEd, edge_len, W_e1, b_e1, W_e2, b_e2,
                W_ef, b_ef, W_es, b_es, bw):
    def body(gs_ref, gd_ref, len_ref, we1_ref, be1_ref,
             we2_ref, be2_ref, wef_ref, bef_ref, wes_ref, bes_ref, bw_ref,
             out_ref):
        bes = _bessel(len_ref[...], bw_ref[...])
        we1 = we1_ref[...]
        gs = gs_ref[...]
        gd = gd_ref[...]
        pre = (gs[:, 0:HID] + gd[:, 0:HID] + _dot(bes, we1[2 * N_NODE:, :])
               + be1_ref[...])
        h = pre * _sigmoid(pre)
        ef = _dot(h, we2_ref[...]) + be2_ref[...]
        wef_mid = wef_ref[...][N_NODE:N_NODE + N_EDGE, :]
        wes_mid = wes_ref[...][N_NODE:N_NODE + N_EDGE, :]
        af = (gs[:, HID:HID + N_EDGE] + gd[:, HID:HID + N_EDGE]
              + _dot(ef, wef_mid) + bef_ref[...])
        ac = (gs[:, HID + N_EDGE:HID + 2 * N_EDGE]
              + gd[:, HID + N_EDGE:HID + 2 * N_EDGE]
              + _dot(ef, wes_mid) + bes_ref[...])
        out_ref[...] = _sigmoid(af) * _softplus(ac)

    wspec = lambda shape: pl.BlockSpec(shape, lambda i: (0, 0))
    return pl.pallas_call(
        body,
        grid=(GRID_P,),
        in_specs=[
            pl.BlockSpec((BE, N_NODE), lambda i: (i, 0)),
            pl.BlockSpec((BE, N_NODE), lambda i: (i, 0)),
            pl.BlockSpec((BE, 1), lambda i: (i, 0)),
            wspec(W_e1.shape), wspec(b_e1.shape),
            wspec(W_e2.shape), wspec(b_e2.shape),
            wspec(W_ef.shape), wspec(b_ef.shape),
            wspec(W_es.shape), wspec(b_es.shape), wspec(bw.shape),
        ],
        out_specs=pl.BlockSpec((BE, N_EDGE), lambda i: (i, 0)),
        out_shape=jax.ShapeDtypeStruct((PART, N_EDGE), _f32),
        name="tc_edge_final",
    )(gEs, gEd, edge_len, W_e1, b_e1, W_e2, b_e2, W_ef, b_ef,
      W_es, b_es, bw)


def kernel(atom_types, env_index, edge_index, env_length, edge_length,
           W_node, b_node, bessel_w, W_e1, b_e1, W_e2, b_e2,
           W_ef, b_ef, W_es, b_es, W_nf, b_nf, W_ns, b_ns, ln_g, ln_b):
    atoms2d = atom_types.astype(jnp.int32).reshape(N, 1)
    env_s = env_index[0].astype(jnp.int32)
    env_d = env_index[1].astype(jnp.int32)
    edge_s = edge_index[0].astype(jnp.int32)
    edge_d = edge_index[1].astype(jnp.int32)
    env_len = env_length.reshape(E, 1)
    edge_len = edge_length.reshape(E, 1)
    bw = bessel_w.reshape(1, N_BASIS)
    b_node2 = b_node.reshape(1, N_NODE)
    b_e1_2 = b_e1.reshape(1, HID)
    b_e2_2 = b_e2.reshape(1, N_EDGE)
    b_ef_2 = b_ef.reshape(1, N_EDGE)
    b_es_2 = b_es.reshape(1, N_EDGE)
    b_nf_2 = b_nf.reshape(1, N_NODE)
    b_ns_2 = b_ns.reshape(1, N_NODE)
    ln_g_2 = ln_g.reshape(1, N_NODE)
    ln_b_2 = ln_b.reshape(1, N_NODE)
    zeros_nf = jnp.zeros((NPAD, N_NODE), _f32)

    node_f, U, V, ts8, td8 = _node_precompute(atoms2d, W_node, b_node2,
                                              W_e1, W_nf, W_ns)
    atoms_i32 = atom_types.astype(jnp.int32)

    msg_parts = []
    for p in range(PARTS):
        sl = slice(p * PART, (p + 1) * PART)
        at_s, at_d = _atom_gather(atoms_i32, env_s[sl], env_d[sl],
                                  "sc_gather_env")
        msg_parts.append(
            _env_msg(at_s.reshape(PART, 1), at_d.reshape(PART, 1),
                     env_len[sl], ts8, td8, W_e1, b_e1_2, W_e2, b_e2_2,
                     W_nf, b_nf_2, W_ns, b_ns_2, ln_g_2, ln_b_2, bw))

    acc = _scatter_add(msg_parts, env_d, zeros_nf)

    node_out, t_es, t_ed = _node_update(node_f, acc, W_ef, W_es, U, V)

    edge_parts = []
    for p in range(PARTS):
        sl = slice(p * PART, (p + 1) * PART)
        gEs, gEd = _dual_gather(t_es, t_ed, edge_s[sl], edge_d[sl],
                                "sc_gather_edge")
        edge_parts.append(
            _edge_final(gEs, gEd, edge_len[sl], W_e1, b_e1_2,
                        W_e2, b_e2_2, W_ef, b_ef_2, W_es, b_es_2, bw))
    edge_out = jnp.concatenate(edge_parts, axis=0)

    return (node_out, edge_out)


# speedup vs baseline: 1.1163x; 1.1163x over previous
import functools

import jax
import jax.numpy as jnp
from jax import lax
from jax.experimental import pallas as pl
from jax.experimental.pallas import tpu as pltpu
from jax.experimental.pallas import tpu_sc as plsc

N = 10000
E = 320000
N_ATOM = 8
N_NODE = 128
N_EDGE = 16
N_BASIS = 8
HID = 64
R_MAX = 5.0

NC = 2
NS = 16
NW = NC * NS
CHUNK = 80
SUPER = 5
PARTS = 5
PART = E // PARTS
PW = PART // NW
PITERS = PW // CHUNK

_f32 = jnp.float32


def _sc_mesh():
    return plsc.VectorSubcoreMesh(core_axis_name="c", subcore_axis_name="s")


def _dual_gather(table_a, table_b, idx_a, idx_b, name):
    D = table_a.shape[1]

    @functools.partial(
        pl.kernel,
        out_type=(
            jax.ShapeDtypeStruct((PART, D), _f32),
            jax.ShapeDtypeStruct((PART, D), _f32),
        ),
        mesh=_sc_mesh(),
        scratch_types=[
            [pltpu.VMEM((CHUNK,), jnp.int32)] * 2,
            [pltpu.VMEM((CHUNK, D), _f32)] * 2,
            pltpu.SemaphoreType.DMA,
            pltpu.SemaphoreType.DMA,
        ],
        name=name,
    )
    def k(ta, tb, ia, ib, oa, ob, idxv, rowv, sema, semb):
        wid = lax.axis_index("s") * NC + lax.axis_index("c")
        base = wid * PW
        iav, ibv = idxv
        rav, rbv = rowv

        def body(i, carry):
            b = base + i * CHUNK
            pltpu.sync_copy(ia.at[pl.ds(b, CHUNK)], iav)
            pltpu.sync_copy(ib.at[pl.ds(b, CHUNK)], ibv)
            cpa = pltpu.async_copy(ta.at[iav], rav, sema)
            cpb = pltpu.async_copy(tb.at[ibv], rbv, semb)
            cpa.wait()
            cpb.wait()
            pltpu.sync_copy(rav, oa.at[pl.ds(b, CHUNK)])
            pltpu.sync_copy(rbv, ob.at[pl.ds(b, CHUNK)])
            return carry

        lax.fori_loop(0, PITERS, body, 0)

    return k(table_a, table_b, idx_a, idx_b)


NPAD = 10240


def _scatter_add(msg_parts, idx_dst, zeros_nf):
    rows_pt = NPAD // NS

    @functools.partial(
        pl.kernel,
        out_type=jax.ShapeDtypeStruct((NC, NPAD, N_NODE), _f32),
        mesh=_sc_mesh(),
        scratch_types=[
            pltpu.VMEM((CHUNK,), jnp.int32),
            pltpu.VMEM((CHUNK, N_NODE), _f32),
            pltpu.VMEM_SHARED((NPAD, N_NODE), _f32),
        ],
        name="sc_scatter_add",
    )
    def k(m0, m1, m2, m3, m4, idx_h, zero_h, out_h, idxv, msgv, acc):
        cid = lax.axis_index("c")
        sid = lax.axis_index("s")
        wid = sid * NC + cid
        pltpu.sync_copy(zero_h.at[pl.ds(sid * rows_pt, rows_pt)],
                        acc.at[pl.ds(sid * rows_pt, rows_pt)])
        plsc.subcore_barrier()

        for p, msg_h in enumerate((m0, m1, m2, m3, m4)):
            def body(i, carry, msg_h=msg_h, p=p):
                bl = wid * PW + i * CHUNK
                pltpu.sync_copy(idx_h.at[pl.ds(p * PART + bl, CHUNK)], idxv)
                pltpu.sync_copy(msg_h.at[pl.ds(bl, CHUNK)], msgv)
                pltpu.sync_copy(msgv, acc.at[idxv], add=True)
                return carry

            lax.fori_loop(0, PITERS, body, 0)
        plsc.subcore_barrier()
        pltpu.sync_copy(acc.at[pl.ds(sid * rows_pt, rows_pt)],
                        out_h.at[cid, pl.ds(sid * rows_pt, rows_pt)])

    return k(*msg_parts, idx_dst, zeros_nf)


def _sigmoid(x):
    return jax.nn.sigmoid(x)


def _softplus(x):
    return jnp.maximum(x, 0.0) + jnp.log1p(jnp.exp(-jnp.abs(x)))


_RN = 12582912.0
_SIN_C = [6.283185302468587, -41.34170039298465, 81.6051308111703,
          -76.70300555852963, 42.02585149179802, -14.899578443268958,
          3.2377800937143286]


def _fast_recip(y):
    i = lax.bitcast_convert_type(y, jnp.int32)
    r = lax.bitcast_convert_type(0x7EF311C3 - i, _f32)
    r = r * (2.0 - y * r)
    r = r * (2.0 - y * r)
    r = r * (2.0 - y * r)
    return r


def _bessel(r, bw):
    u = bw * (r * float(1.0 / (2.0 * 3.141592653589793 * R_MAX)))
    n = (u + _RN) - _RN
    f = u - n
    f2 = f * f
    acc = f2 * _SIN_C[-1] + _SIN_C[-2]
    for v in _SIN_C[-3::-1]:
        acc = acc * f2 + v
    return (2.0 / R_MAX) ** 0.5 * (f * acc) * _fast_recip(r)


def _dot(a, b):
    return jax.lax.dot_general(a, b, (((1,), (0,)), ((), ())),
                               preferred_element_type=_f32)


def _node_precompute(atoms2d, W_node, b_node, W_e1):
    def body(at_ref, wn_ref, bn_ref, we1_ref, nf_ref, u_ref, v_ref):
        at = at_ref[...]
        oh = (at == lax.broadcasted_iota(jnp.int32, (N, N_ATOM), 1)).astype(_f32)
        nf = _dot(oh, wn_ref[...]) + bn_ref[...]
        nf_ref[...] = nf
        we1 = we1_ref[...]
        u_ref[...] = _dot(nf, we1[0:N_NODE, :])
        v_ref[...] = _dot(nf, we1[N_NODE:2 * N_NODE, :])

    return pl.pallas_call(
        body,
        out_shape=(
            jax.ShapeDtypeStruct((N, N_NODE), _f32),
            jax.ShapeDtypeStruct((N, HID), _f32),
            jax.ShapeDtypeStruct((N, HID), _f32),
        ),
        name="tc_node_precompute",
    )(atoms2d, W_node, b_node, W_e1)


BE = 4000
GRID_P = PART // BE


def _env_msg(gs, gd, env_len, W_e1, b_e1, W_e2, b_e2, W_nf, b_nf,
             W_ns, b_ns, ln_g, ln_b, bw):
    def body(gs_ref, gd_ref, len_ref, we1_ref, be1_ref, we2_ref, be2_ref,
             wnf_ref, bnf_ref, wns_ref, bns_ref, g_ref, bln_ref, bw_ref,
             msg_ref):
        s = gs_ref[...]
        d = gd_ref[...]
        bes = _bessel(len_ref[...], bw_ref[...])
        we1 = we1_ref[...]
        pre = (_dot(s, we1[0:N_NODE, :]) + _dot(d, we1[N_NODE:2 * N_NODE, :])
               + _dot(bes, we1[2 * N_NODE:, :]) + be1_ref[...])
        h = pre * _sigmoid(pre)
        env_f = _dot(h, we2_ref[...]) + be2_ref[...]
        wnf = wnf_ref[...]
        wns = wns_ref[...]
        a = (_dot(s, wnf[0:N_NODE, :]) + _dot(d, wnf[N_NODE:2 * N_NODE, :])
             + _dot(env_f, wnf[2 * N_NODE:, :]) + bnf_ref[...])
        c = (_dot(s, wns[0:N_NODE, :]) + _dot(d, wns[N_NODE:2 * N_NODE, :])
             + _dot(env_f, wns[2 * N_NODE:, :]) + bns_ref[...])
        msg = _sigmoid(a) * _softplus(c)
        mu = jnp.mean(msg, axis=-1, keepdims=True)
        xc = msg - mu
        var = jnp.mean(xc * xc, axis=-1, keepdims=True)
        msg_ref[...] = xc * jax.lax.rsqrt(var + 1e-5) * g_ref[...] + bln_ref[...]

    wspec = lambda shape: pl.BlockSpec(shape, lambda i: (0, 0))
    return pl.pallas_call(
        body,
        grid=(GRID_P,),
        in_specs=[
            pl.BlockSpec((BE, N_NODE), lambda i: (i, 0)),
            pl.BlockSpec((BE, N_NODE), lambda i: (i, 0)),
            pl.BlockSpec((BE, 1), lambda i: (i, 0)),
            wspec(W_e1.shape), wspec(b_e1.shape),
            wspec(W_e2.shape), wspec(b_e2.shape),
            wspec(W_nf.shape), wspec(b_nf.shape),
            wspec(W_ns.shape), wspec(b_ns.shape),
            wspec(ln_g.shape), wspec(ln_b.shape), wspec(bw.shape),
        ],
        out_specs=pl.BlockSpec((BE, N_NODE), lambda i: (i, 0)),
        out_shape=jax.ShapeDtypeStruct((PART, N_NODE), _f32),
        name="tc_env_msg",
    )(gs, gd, env_len, W_e1, b_e1, W_e2, b_e2, W_nf, b_nf, W_ns, b_ns,
      ln_g, ln_b, bw)


def _node_update(node_f, acc, W_ef, W_es, U, V):
    def body(nf_ref, acc_ref, wef_ref, wes_ref, u_ref, v_ref,
             out_ref, ts_ref, td_ref):
        accv = acc_ref[...]
        nf = nf_ref[...] + accv[0, 0:N, :] + accv[1, 0:N, :]
        out_ref[...] = nf
        wef = wef_ref[...]
        wes = wes_ref[...]
        ts_ref[:, 0:HID] = u_ref[...]
        ts_ref[:, HID:HID + N_EDGE] = _dot(nf, wef[0:N_NODE, :])
        ts_ref[:, HID + N_EDGE:HID + 2 * N_EDGE] = _dot(nf, wes[0:N_NODE, :])
        ts_ref[:, HID + 2 * N_EDGE:] = jnp.zeros((N, 32), _f32)
        td_ref[:, 0:HID] = v_ref[...]
        td_ref[:, HID:HID + N_EDGE] = _dot(nf, wef[N_NODE + N_EDGE:, :])
        td_ref[:, HID + N_EDGE:HID + 2 * N_EDGE] = _dot(nf, wes[N_NODE + N_EDGE:, :])
        td_ref[:, HID + 2 * N_EDGE:] = jnp.zeros((N, 32), _f32)

    return pl.pallas_call(
        body,
        out_shape=(
            jax.ShapeDtypeStruct((N, N_NODE), _f32),
            jax.ShapeDtypeStruct((N, N_NODE), _f32),
            jax.ShapeDtypeStruct((N, N_NODE), _f32),
        ),
        name="tc_node_update",
    )(node_f, acc, W_ef, W_es, U, V)


def _edge_final(gEs, gEd, edge_len, W_e1, b_e1, W_e2, b_e2,
                W_ef, b_ef, W_es, b_es, bw):
    def body(gs_ref, gd_ref, len_ref, we1_ref, be1_ref,
             we2_ref, be2_ref, wef_ref, bef_ref, wes_ref, bes_ref, bw_ref,
             out_ref):
        bes = _bessel(len_ref[...], bw_ref[...])
        we1 = we1_ref[...]
        gs = gs_ref[...]
        gd = gd_ref[...]
        pre = (gs[:, 0:HID] + gd[:, 0:HID] + _dot(bes, we1[2 * N_NODE:, :])
               + be1_ref[...])
        h = pre * _sigmoid(pre)
        ef = _dot(h, we2_ref[...]) + be2_ref[...]
        wef_mid = wef_ref[...][N_NODE:N_NODE + N_EDGE, :]
        wes_mid = wes_ref[...][N_NODE:N_NODE + N_EDGE, :]
        af = (gs[:, HID:HID + N_EDGE] + gd[:, HID:HID + N_EDGE]
              + _dot(ef, wef_mid) + bef_ref[...])
        ac = (gs[:, HID + N_EDGE:HID + 2 * N_EDGE]
              + gd[:, HID + N_EDGE:HID + 2 * N_EDGE]
              + _dot(ef, wes_mid) + bes_ref[...])
        out_ref[...] = _sigmoid(af) * _softplus(ac)

    wspec = lambda shape: pl.BlockSpec(shape, lambda i: (0, 0))
    return pl.pallas_call(
        body,
        grid=(GRID_P,),
        in_specs=[
            pl.BlockSpec((BE, N_NODE), lambda i: (i, 0)),
            pl.BlockSpec((BE, N_NODE), lambda i: (i, 0)),
            pl.BlockSpec((BE, 1), lambda i: (i, 0)),
            wspec(W_e1.shape), wspec(b_e1.shape),
            wspec(W_e2.shape), wspec(b_e2.shape),
            wspec(W_ef.shape), wspec(b_ef.shape),
            wspec(W_es.shape), wspec(b_es.shape), wspec(bw.shape),
        ],
        out_specs=pl.BlockSpec((BE, N_EDGE), lambda i: (i, 0)),
        out_shape=jax.ShapeDtypeStruct((PART, N_EDGE), _f32),
        name="tc_edge_final",
    )(gEs, gEd, edge_len, W_e1, b_e1, W_e2, b_e2, W_ef, b_ef,
      W_es, b_es, bw)


def kernel(atom_types, env_index, edge_index, env_length, edge_length,
           W_node, b_node, bessel_w, W_e1, b_e1, W_e2, b_e2,
           W_ef, b_ef, W_es, b_es, W_nf, b_nf, W_ns, b_ns, ln_g, ln_b):
    atoms2d = atom_types.astype(jnp.int32).reshape(N, 1)
    env_s = env_index[0].astype(jnp.int32)
    env_d = env_index[1].astype(jnp.int32)
    edge_s = edge_index[0].astype(jnp.int32)
    edge_d = edge_index[1].astype(jnp.int32)
    env_len = env_length.reshape(E, 1)
    edge_len = edge_length.reshape(E, 1)
    bw = bessel_w.reshape(1, N_BASIS)
    b_node2 = b_node.reshape(1, N_NODE)
    b_e1_2 = b_e1.reshape(1, HID)
    b_e2_2 = b_e2.reshape(1, N_EDGE)
    b_ef_2 = b_ef.reshape(1, N_EDGE)
    b_es_2 = b_es.reshape(1, N_EDGE)
    b_nf_2 = b_nf.reshape(1, N_NODE)
    b_ns_2 = b_ns.reshape(1, N_NODE)
    ln_g_2 = ln_g.reshape(1, N_NODE)
    ln_b_2 = ln_b.reshape(1, N_NODE)
    zeros_nf = jnp.zeros((NPAD, N_NODE), _f32)

    node_f, U, V = _node_precompute(atoms2d, W_node, b_node2, W_e1)

    msg_parts = []
    for p in range(PARTS):
        sl = slice(p * PART, (p + 1) * PART)
        gs, gd = _dual_gather(node_f, node_f, env_s[sl], env_d[sl],
                              "sc_gather_env")
        msg_parts.append(
            _env_msg(gs, gd, env_len[sl], W_e1, b_e1_2, W_e2, b_e2_2,
                     W_nf, b_nf_2, W_ns, b_ns_2, ln_g_2, ln_b_2, bw))

    acc = _scatter_add(msg_parts, env_d, zeros_nf)

    node_out, t_es, t_ed = _node_update(node_f, acc, W_ef, W_es, U, V)

    edge_parts = []
    for p in range(PARTS):
        sl = slice(p * PART, (p + 1) * PART)
        gEs, gEd = _dual_gather(t_es, t_ed, edge_s[sl], edge_d[sl],
                                "sc_gather_edge")
        edge_parts.append(
            _edge_final(gEs, gEd, edge_len[sl], W_e1, b_e1_2,
                        W_e2, b_e2_2, W_ef, b_ef_2, W_es, b_es_2, bw))
    edge_out = jnp.concatenate(edge_parts, axis=0)

    return (node_out, edge_out)


# speedup vs baseline: 1.1372x; 1.0187x over previous
import functools

import jax
import jax.numpy as jnp
from jax import lax
from jax.experimental import pallas as pl
from jax.experimental.pallas import tpu as pltpu
from jax.experimental.pallas import tpu_sc as plsc

N = 10000
E = 320000
N_ATOM = 8
N_NODE = 128
N_EDGE = 16
N_BASIS = 8
HID = 64
R_MAX = 5.0

NC = 2
NS = 16
NW = NC * NS
CHUNK = 80
SUPER = 5
PART_SIZES = (128000, 128000, 64000)
PART_OFFS = (0, 128000, 256000)

_f32 = jnp.float32


def _sc_mesh():
    return plsc.VectorSubcoreMesh(core_axis_name="c", subcore_axis_name="s")


def _dual_gather(table_a, table_b, idx_a, idx_b, name):
    D = table_a.shape[1]
    ne = idx_a.shape[0]
    pw = ne // NW

    @functools.partial(
        pl.kernel,
        out_type=(
            jax.ShapeDtypeStruct((ne, D), _f32),
            jax.ShapeDtypeStruct((ne, D), _f32),
        ),
        mesh=_sc_mesh(),
        scratch_types=[
            [pltpu.VMEM((CHUNK,), jnp.int32)] * 2,
            [pltpu.VMEM((CHUNK, D), _f32)] * 2,
            pltpu.SemaphoreType.DMA,
            pltpu.SemaphoreType.DMA,
        ],
        name=name,
    )
    def k(ta, tb, ia, ib, oa, ob, idxv, rowv, sema, semb):
        wid = lax.axis_index("s") * NC + lax.axis_index("c")
        base = wid * pw
        iav, ibv = idxv
        rav, rbv = rowv

        def body(i, carry):
            b = base + i * CHUNK
            pltpu.sync_copy(ia.at[pl.ds(b, CHUNK)], iav)
            pltpu.sync_copy(ib.at[pl.ds(b, CHUNK)], ibv)
            cpa = pltpu.async_copy(ta.at[iav], rav, sema)
            cpb = pltpu.async_copy(tb.at[ibv], rbv, semb)
            cpa.wait()
            cpb.wait()
            pltpu.sync_copy(rav, oa.at[pl.ds(b, CHUNK)])
            pltpu.sync_copy(rbv, ob.at[pl.ds(b, CHUNK)])
            return carry

        lax.fori_loop(0, pw // CHUNK, body, 0)

    return k(table_a, table_b, idx_a, idx_b)


NPAD = 10240


def _scatter_add(msg_parts, idx_dst, zeros_nf):
    rows_pt = NPAD // NS

    @functools.partial(
        pl.kernel,
        out_type=jax.ShapeDtypeStruct((NC, NPAD, N_NODE), _f32),
        mesh=_sc_mesh(),
        scratch_types=[
            pltpu.VMEM((CHUNK,), jnp.int32),
            pltpu.VMEM((CHUNK, N_NODE), _f32),
            pltpu.VMEM_SHARED((NPAD, N_NODE), _f32),
        ],
        name="sc_scatter_add",
    )
    def k(m0, m1, m2, idx_h, zero_h, out_h, idxv, msgv, acc):
        cid = lax.axis_index("c")
        sid = lax.axis_index("s")
        wid = sid * NC + cid
        pltpu.sync_copy(zero_h.at[pl.ds(sid * rows_pt, rows_pt)],
                        acc.at[pl.ds(sid * rows_pt, rows_pt)])
        plsc.subcore_barrier()

        for p, msg_h in enumerate((m0, m1, m2)):
            pw = PART_SIZES[p] // NW

            def body(i, carry, msg_h=msg_h, p=p, pw=pw):
                bl = wid * pw + i * CHUNK
                pltpu.sync_copy(idx_h.at[pl.ds(PART_OFFS[p] + bl, CHUNK)], idxv)
                pltpu.sync_copy(msg_h.at[pl.ds(bl, CHUNK)], msgv)
                pltpu.sync_copy(msgv, acc.at[idxv], add=True)
                return carry

            lax.fori_loop(0, PART_SIZES[p] // NW // CHUNK, body, 0)
        plsc.subcore_barrier()
        pltpu.sync_copy(acc.at[pl.ds(sid * rows_pt, rows_pt)],
                        out_h.at[cid, pl.ds(sid * rows_pt, rows_pt)])

    return k(*msg_parts, idx_dst, zeros_nf)


def _sigmoid(x):
    return jax.nn.sigmoid(x)


def _softplus(x):
    return jnp.maximum(x, 0.0) + jnp.log1p(jnp.exp(-jnp.abs(x)))


_RN = 12582912.0
_SIN_C = [6.283185302468587, -41.34170039298465, 81.6051308111703,
          -76.70300555852963, 42.02585149179802, -14.899578443268958,
          3.2377800937143286]


def _fast_recip(y):
    i = lax.bitcast_convert_type(y, jnp.int32)
    r = lax.bitcast_convert_type(0x7EF311C3 - i, _f32)
    r = r * (2.0 - y * r)
    r = r * (2.0 - y * r)
    r = r * (2.0 - y * r)
    return r


def _bessel(r, bw):
    u = bw * (r * float(1.0 / (2.0 * 3.141592653589793 * R_MAX)))
    n = (u + _RN) - _RN
    f = u - n
    f2 = f * f
    acc = f2 * _SIN_C[-1] + _SIN_C[-2]
    for v in _SIN_C[-3::-1]:
        acc = acc * f2 + v
    return (2.0 / R_MAX) ** 0.5 * (f * acc) * _fast_recip(r)


def _dot(a, b):
    return jax.lax.dot_general(a, b, (((1,), (0,)), ((), ())),
                               preferred_element_type=_f32)


def _node_precompute(atoms2d, W_node, b_node, W_e1):
    def body(at_ref, wn_ref, bn_ref, we1_ref, nf_ref, u_ref, v_ref):
        at = at_ref[...]
        oh = (at == lax.broadcasted_iota(jnp.int32, (N, N_ATOM), 1)).astype(_f32)
        nf = _dot(oh, wn_ref[...]) + bn_ref[...]
        nf_ref[...] = nf
        we1 = we1_ref[...]
        u_ref[...] = _dot(nf, we1[0:N_NODE, :])
        v_ref[...] = _dot(nf, we1[N_NODE:2 * N_NODE, :])

    return pl.pallas_call(
        body,
        out_shape=(
            jax.ShapeDtypeStruct((N, N_NODE), _f32),
            jax.ShapeDtypeStruct((N, HID), _f32),
            jax.ShapeDtypeStruct((N, HID), _f32),
        ),
        name="tc_node_precompute",
    )(atoms2d, W_node, b_node, W_e1)


BE = 4000


def _env_msg(gs, gd, env_len, W_e1, b_e1, W_e2, b_e2, W_nf, b_nf,
             W_ns, b_ns, ln_g, ln_b, bw):
    def body(gs_ref, gd_ref, len_ref, we1_ref, be1_ref, we2_ref, be2_ref,
             wnf_ref, bnf_ref, wns_ref, bns_ref, g_ref, bln_ref, bw_ref,
             msg_ref):
        s = gs_ref[...]
        d = gd_ref[...]
        bes = _bessel(len_ref[...], bw_ref[...])
        we1 = we1_ref[...]
        pre = (_dot(s, we1[0:N_NODE, :]) + _dot(d, we1[N_NODE:2 * N_NODE, :])
               + _dot(bes, we1[2 * N_NODE:, :]) + be1_ref[...])
        h = pre * _sigmoid(pre)
        env_f = _dot(h, we2_ref[...]) + be2_ref[...]
        wnf = wnf_ref[...]
        wns = wns_ref[...]
        a = (_dot(s, wnf[0:N_NODE, :]) + _dot(d, wnf[N_NODE:2 * N_NODE, :])
             + _dot(env_f, wnf[2 * N_NODE:, :]) + bnf_ref[...])
        c = (_dot(s, wns[0:N_NODE, :]) + _dot(d, wns[N_NODE:2 * N_NODE, :])
             + _dot(env_f, wns[2 * N_NODE:, :]) + bns_ref[...])
        msg = _sigmoid(a) * _softplus(c)
        mu = jnp.mean(msg, axis=-1, keepdims=True)
        xc = msg - mu
        var = jnp.mean(xc * xc, axis=-1, keepdims=True)
        msg_ref[...] = xc * jax.lax.rsqrt(var + 1e-5) * g_ref[...] + bln_ref[...]

    wspec = lambda shape: pl.BlockSpec(shape, lambda i: (0, 0))
    return pl.pallas_call(
        body,
        grid=(gs.shape[0] // BE,),
        in_specs=[
            pl.BlockSpec((BE, N_NODE), lambda i: (i, 0)),
            pl.BlockSpec((BE, N_NODE), lambda i: (i, 0)),
            pl.BlockSpec((BE, 1), lambda i: (i, 0)),
            wspec(W_e1.shape), wspec(b_e1.shape),
            wspec(W_e2.shape), wspec(b_e2.shape),
            wspec(W_nf.shape), wspec(b_nf.shape),
            wspec(W_ns.shape), wspec(b_ns.shape),
            wspec(ln_g.shape), wspec(ln_b.shape), wspec(bw.shape),
        ],
        out_specs=pl.BlockSpec((BE, N_NODE), lambda i: (i, 0)),
        out_shape=jax.ShapeDtypeStruct((gs.shape[0], N_NODE), _f32),
        name="tc_env_msg",
    )(gs, gd, env_len, W_e1, b_e1, W_e2, b_e2, W_nf, b_nf, W_ns, b_ns,
      ln_g, ln_b, bw)


def _node_update(node_f, acc, W_ef, W_es, U, V):
    def body(nf_ref, acc_ref, wef_ref, wes_ref, u_ref, v_ref,
             out_ref, ts_ref, td_ref):
        accv = acc_ref[...]
        nf = nf_ref[...] + accv[0, 0:N, :] + accv[1, 0:N, :]
        out_ref[...] = nf
        wef = wef_ref[...]
        wes = wes_ref[...]
        ts_ref[:, 0:HID] = u_ref[...]
        ts_ref[:, HID:HID + N_EDGE] = _dot(nf, wef[0:N_NODE, :])
        ts_ref[:, HID + N_EDGE:HID + 2 * N_EDGE] = _dot(nf, wes[0:N_NODE, :])
        ts_ref[:, HID + 2 * N_EDGE:] = jnp.zeros((N, 32), _f32)
        td_ref[:, 0:HID] = v_ref[...]
        td_ref[:, HID:HID + N_EDGE] = _dot(nf, wef[N_NODE + N_EDGE:, :])
        td_ref[:, HID + N_EDGE:HID + 2 * N_EDGE] = _dot(nf, wes[N_NODE + N_EDGE:, :])
        td_ref[:, HID + 2 * N_EDGE:] = jnp.zeros((N, 32), _f32)

    return pl.pallas_call(
        body,
        out_shape=(
            jax.ShapeDtypeStruct((N, N_NODE), _f32),
            jax.ShapeDtypeStruct((N, N_NODE), _f32),
            jax.ShapeDtypeStruct((N, N_NODE), _f32),
        ),
        name="tc_node_update",
    )(node_f, acc, W_ef, W_es, U, V)


def _edge_final(gEs, gEd, edge_len, W_e1, b_e1, W_e2, b_e2,
                W_ef, b_ef, W_es, b_es, bw):
    def body(gs_ref, gd_ref, len_ref, we1_ref, be1_ref,
             we2_ref, be2_ref, wef_ref, bef_ref, wes_ref, bes_ref, bw_ref,
             out_ref):
        bes = _bessel(len_ref[...], bw_ref[...])
        we1 = we1_ref[...]
        gs = gs_ref[...]
        gd = gd_ref[...]
        pre = (gs[:, 0:HID] + gd[:, 0:HID] + _dot(bes, we1[2 * N_NODE:, :])
               + be1_ref[...])
        h = pre * _sigmoid(pre)
        ef = _dot(h, we2_ref[...]) + be2_ref[...]
        wef_mid = wef_ref[...][N_NODE:N_NODE + N_EDGE, :]
        wes_mid = wes_ref[...][N_NODE:N_NODE + N_EDGE, :]
        af = (gs[:, HID:HID + N_EDGE] + gd[:, HID:HID + N_EDGE]
              + _dot(ef, wef_mid) + bef_ref[...])
        ac = (gs[:, HID + N_EDGE:HID + 2 * N_EDGE]
              + gd[:, HID + N_EDGE:HID + 2 * N_EDGE]
              + _dot(ef, wes_mid) + bes_ref[...])
        out_ref[...] = _sigmoid(af) * _softplus(ac)

    wspec = lambda shape: pl.BlockSpec(shape, lambda i: (0, 0))
    return pl.pallas_call(
        body,
        grid=(gEs.shape[0] // BE,),
        in_specs=[
            pl.BlockSpec((BE, N_NODE), lambda i: (i, 0)),
            pl.BlockSpec((BE, N_NODE), lambda i: (i, 0)),
            pl.BlockSpec((BE, 1), lambda i: (i, 0)),
            wspec(W_e1.shape), wspec(b_e1.shape),
            wspec(W_e2.shape), wspec(b_e2.shape),
            wspec(W_ef.shape), wspec(b_ef.shape),
            wspec(W_es.shape), wspec(b_es.shape), wspec(bw.shape),
        ],
        out_specs=pl.BlockSpec((BE, N_EDGE), lambda i: (i, 0)),
        out_shape=jax.ShapeDtypeStruct((gEs.shape[0], N_EDGE), _f32),
        name="tc_edge_final",
    )(gEs, gEd, edge_len, W_e1, b_e1, W_e2, b_e2, W_ef, b_ef,
      W_es, b_es, bw)


def kernel(atom_types, env_index, edge_index, env_length, edge_length,
           W_node, b_node, bessel_w, W_e1, b_e1, W_e2, b_e2,
           W_ef, b_ef, W_es, b_es, W_nf, b_nf, W_ns, b_ns, ln_g, ln_b):
    atoms2d = atom_types.astype(jnp.int32).reshape(N, 1)
    env_s = env_index[0].astype(jnp.int32)
    env_d = env_index[1].astype(jnp.int32)
    edge_s = edge_index[0].astype(jnp.int32)
    edge_d = edge_index[1].astype(jnp.int32)
    env_len = env_length.reshape(E, 1)
    edge_len = edge_length.reshape(E, 1)
    bw = bessel_w.reshape(1, N_BASIS)
    b_node2 = b_node.reshape(1, N_NODE)
    b_e1_2 = b_e1.reshape(1, HID)
    b_e2_2 = b_e2.reshape(1, N_EDGE)
    b_ef_2 = b_ef.reshape(1, N_EDGE)
    b_es_2 = b_es.reshape(1, N_EDGE)
    b_nf_2 = b_nf.reshape(1, N_NODE)
    b_ns_2 = b_ns.reshape(1, N_NODE)
    ln_g_2 = ln_g.reshape(1, N_NODE)
    ln_b_2 = ln_b.reshape(1, N_NODE)
    zeros_nf = jnp.zeros((NPAD, N_NODE), _f32)

    node_f, U, V = _node_precompute(atoms2d, W_node, b_node2, W_e1)

    msg_parts = []
    for p, (off, ne) in enumerate(zip(PART_OFFS, PART_SIZES)):
        sl = slice(off, off + ne)
        gs, gd = _dual_gather(node_f, node_f, env_s[sl], env_d[sl],
                              "sc_gather_env")
        msg_parts.append(
            _env_msg(gs, gd, env_len[sl], W_e1, b_e1_2, W_e2, b_e2_2,
                     W_nf, b_nf_2, W_ns, b_ns_2, ln_g_2, ln_b_2, bw))

    acc = _scatter_add(msg_parts, env_d, zeros_nf)

    node_out, t_es, t_ed = _node_update(node_f, acc, W_ef, W_es, U, V)

    edge_parts = []
    for p, (off, ne) in enumerate(zip(PART_OFFS, PART_SIZES)):
        sl = slice(off, off + ne)
        gEs, gEd = _dual_gather(t_es, t_ed, edge_s[sl], edge_d[sl],
                                "sc_gather_edge")
        edge_parts.append(
            _edge_final(gEs, gEd, edge_len[sl], W_e1, b_e1_2,
                        W_e2, b_e2_2, W_ef, b_ef_2, W_es, b_es_2, bw))
    edge_out = jnp.concatenate(edge_parts, axis=0)

    return (node_out, edge_out)
